# baseline (speedup 1.0000x reference)
import functools

import jax
import jax.numpy as jnp
from jax import lax
from jax.experimental import pallas as pl
from jax.experimental.pallas import tpu as pltpu
from jax.experimental.pallas import tpu_sc as plsc

ROWS = 8192
COLS = 2048
REP = 4
NC = 2
NS = 16
NW = NC * NS
SC_ROWS = 3072
ROWS_PER_W = SC_ROWS // NW
CH = 32
NCHUNK = ROWS_PER_W // CH

TC_ROWS = ROWS - SC_ROWS
TC_BLK = 128


def _sc_kernel(x_hbm, out_hbm, buf, idx0, idx1, idx2, idx3, sem):
    wid = lax.axis_index("s") * NC + lax.axis_index("c")
    base0 = wid * ROWS_PER_W
    idx_refs = (idx0, idx1, idx2, idx3)

    def chunk_body(g, carry):
        base = base0 + g * CH
        pltpu.sync_copy(x_hbm.at[pl.ds(base, CH)], buf)
        for t in range(CH // 16):
            rows = base + t * 16 + lax.iota(jnp.int32, 16)
            for j in range(REP):
                idx_refs[j][pl.ds(t * 16, 16)] = rows * REP + j
        copies = [
            pltpu.async_copy(buf, out_hbm.at[idx_refs[j]], sem)
            for j in range(REP)
        ]
        for c in copies:
            c.wait()
        return carry

    lax.fori_loop(0, NCHUNK, chunk_body, 0)


def _tc_body(x_ref, o_ref):
    o_ref[...] = jnp.repeat(x_ref[...], REP, axis=0)


@jax.jit
def _repeat_interleave(x):
    mesh = plsc.VectorSubcoreMesh(core_axis_name="c", subcore_axis_name="s")
    sc = functools.partial(
        pl.kernel,
        out_type=jax.ShapeDtypeStruct((SC_ROWS * REP, COLS), jnp.float32),
        mesh=mesh,
        scratch_types=[
            pltpu.VMEM((CH, COLS), jnp.float32),
            pltpu.VMEM((CH,), jnp.int32),
            pltpu.VMEM((CH,), jnp.int32),
            pltpu.VMEM((CH,), jnp.int32),
            pltpu.VMEM((CH,), jnp.int32),
            pltpu.SemaphoreType.DMA,
        ],
    )(_sc_kernel)
    sc_out = sc(x)

    tc_out = pl.pallas_call(
        _tc_body,
        grid=(TC_ROWS // TC_BLK,),
        in_specs=[
            pl.BlockSpec((TC_BLK, COLS), lambda i: (SC_ROWS // TC_BLK + i, 0))
        ],
        out_specs=pl.BlockSpec((TC_BLK * REP, COLS), lambda i: (i, 0)),
        out_shape=jax.ShapeDtypeStruct((TC_ROWS * REP, COLS), jnp.float32),
    )(x)

    return jnp.concatenate([sc_out, tc_out], axis=0)


def kernel(x):
    return _repeat_interleave(x)


# speedup vs baseline: 2.1077x; 2.1077x over previous
import functools

import jax
import jax.numpy as jnp
from jax import lax
from jax.experimental import pallas as pl
from jax.experimental.pallas import tpu as pltpu
from jax.experimental.pallas import tpu_sc as plsc

ROWS = 8192
COLS = 2048
REP = 4
NC = 2
NS = 16
NW = NC * NS
ROWS_PER_W = ROWS // NW
CH = 16
NCHUNK = ROWS_PER_W // CH
NPAIR = NCHUNK // 2


def _sc_kernel(x_hbm, out_hbm,
               b0, b1, i00, i01, i02, i03, i10, i11, i12, i13,
               gs0, gs1, ss0, ss1):
    wid = lax.axis_index("s") * NC + lax.axis_index("c")
    base0 = wid * ROWS_PER_W
    slots = (
        (b0, (i00, i01, i02, i03), gs0, ss0),
        (b1, (i10, i11, i12, i13), gs1, ss1),
    )

    pltpu.async_copy(x_hbm.at[pl.ds(base0, CH)], b0, gs0)
    pltpu.async_copy(x_hbm.at[pl.ds(base0 + CH, CH)], b1, gs1)

    def pair_body(p, carry):
        for s in range(2):
            g = 2 * p + s
            buf, idxs, gsem, ssem = slots[s]
            base = base0 + g * CH
            pltpu.make_async_copy(x_hbm.at[pl.ds(base, CH)], buf, gsem).wait()
            rows = base + lax.iota(jnp.int32, CH)
            for j in range(REP):
                idxs[j][...] = rows * REP + j
            for j in range(REP):
                pltpu.async_copy(buf, out_hbm.at[idxs[j]], ssem)
        for s in range(2):
            g = 2 * p + s
            buf, idxs, gsem, ssem = slots[s]
            for j in range(REP):
                pltpu.make_async_copy(buf, out_hbm.at[idxs[j]], ssem).wait()
            nxt = base0 + (g + 2) * CH

            @pl.when(g + 2 < NCHUNK)
            def _():
                pltpu.async_copy(x_hbm.at[pl.ds(nxt, CH)], buf, gsem)

        return carry

    lax.fori_loop(0, NPAIR, pair_body, 0)


@jax.jit
def _repeat_interleave(x):
    mesh = plsc.VectorSubcoreMesh(core_axis_name="c", subcore_axis_name="s")
    k = functools.partial(
        pl.kernel,
        out_type=jax.ShapeDtypeStruct((ROWS * REP, COLS), jnp.float32),
        mesh=mesh,
        scratch_types=[
            pltpu.VMEM((CH, COLS), jnp.float32),
            pltpu.VMEM((CH, COLS), jnp.float32),
            pltpu.VMEM((CH,), jnp.int32),
            pltpu.VMEM((CH,), jnp.int32),
            pltpu.VMEM((CH,), jnp.int32),
            pltpu.VMEM((CH,), jnp.int32),
            pltpu.VMEM((CH,), jnp.int32),
            pltpu.VMEM((CH,), jnp.int32),
            pltpu.VMEM((CH,), jnp.int32),
            pltpu.VMEM((CH,), jnp.int32),
            pltpu.SemaphoreType.DMA,
            pltpu.SemaphoreType.DMA,
            pltpu.SemaphoreType.DMA,
            pltpu.SemaphoreType.DMA,
        ],
    )(_sc_kernel)
    return k(x)


def kernel(x):
    return _repeat_interleave(x)


# speedup vs baseline: 2.1079x; 1.0001x over previous
import functools

import jax
import jax.numpy as jnp
from jax import lax
from jax.experimental import pallas as pl
from jax.experimental.pallas import tpu as pltpu
from jax.experimental.pallas import tpu_sc as plsc

ROWS = 8192
COLS = 2048
REP = 4
NC = 2
NS = 16
NW = NC * NS
ROWS_PER_W = ROWS // NW
CH = 32
NCHUNK = ROWS_PER_W // CH


def _repeat_kernel(x_hbm, out_hbm, buf, idx0, idx1, idx2, idx3, sem):
    wid = lax.axis_index("s") * NC + lax.axis_index("c")
    base0 = wid * ROWS_PER_W
    idx_refs = (idx0, idx1, idx2, idx3)

    def chunk_body(g, carry):
        base = base0 + g * CH
        pltpu.sync_copy(x_hbm.at[pl.ds(base, CH)], buf)
        for t in range(CH // 16):
            rows = base + t * 16 + lax.iota(jnp.int32, 16)
            for j in range(REP):
                idx_refs[j][pl.ds(t * 16, 16)] = rows * REP + j
        copies = [
            pltpu.async_copy(buf, out_hbm.at[idx_refs[j]], sem)
            for j in range(REP)
        ]
        for c in copies:
            c.wait()
        return carry

    lax.fori_loop(0, NCHUNK, chunk_body, 0)


@jax.jit
def _repeat_interleave(x):
    mesh = plsc.VectorSubcoreMesh(core_axis_name="c", subcore_axis_name="s")
    k = functools.partial(
        pl.kernel,
        out_type=jax.ShapeDtypeStruct((ROWS * REP, COLS), jnp.float32),
        mesh=mesh,
        scratch_types=[
            pltpu.VMEM((CH, COLS), jnp.float32),
            pltpu.VMEM((CH,), jnp.int32),
            pltpu.VMEM((CH,), jnp.int32),
            pltpu.VMEM((CH,), jnp.int32),
            pltpu.VMEM((CH,), jnp.int32),
            pltpu.SemaphoreType.DMA,
        ],
    )(_repeat_kernel)
    return k(x)


def kernel(x):
    return _repeat_interleave(x)
